# baseline (speedup 1.0000x reference)
import jax
import jax.numpy as jnp
from jax.experimental import pallas as pl
from jax.experimental.pallas import tpu as pltpu

_C = 16
_NBUF = 6


def _body(f_hbm, m_hbm, e_hbm, o_hbm, in_buf, out_buf, pe_buf, mask_buf,
          in_sems, out_sems, aux_sem):
    B, L, D = f_hbm.shape
    nchunk = B // _C

    emb_cp = pltpu.make_async_copy(e_hbm, pe_buf, aux_sem)
    emb_cp.start()
    mask_cp = pltpu.make_async_copy(m_hbm, mask_buf, aux_sem)
    mask_cp.start()
    emb_cp.wait()
    mask_cp.wait()
    pe = jnp.maximum(pe_buf[...], 0.0)

    def read(i, slot):
        pltpu.make_async_copy(
            f_hbm.at[pl.ds(i * _C, _C)], in_buf.at[slot], in_sems.at[slot]
        ).start()

    def wait_read(i, slot):
        pltpu.make_async_copy(
            f_hbm.at[pl.ds(i * _C, _C)], in_buf.at[slot], in_sems.at[slot]
        ).wait()

    def write(i, slot):
        pltpu.make_async_copy(
            out_buf.at[slot], o_hbm.at[pl.ds(i * _C, _C)], out_sems.at[slot]
        ).start()

    def wait_write(i, slot):
        pltpu.make_async_copy(
            out_buf.at[slot], o_hbm.at[pl.ds(i * _C, _C)], out_sems.at[slot]
        ).wait()

    for i in range(min(_NBUF, nchunk)):
        read(i, i % _NBUF)

    for i in range(nchunk):
        slot = i % _NBUF
        if i >= _NBUF:
            wait_write(i - _NBUF, slot)
        wait_read(i, slot)
        mk = mask_buf[pl.ds(i * _C, _C), :]
        out_buf[slot] = in_buf[slot] + pe[None, :, :] * mk[:, :, None]
        write(i, slot)
        nxt = i + _NBUF
        if nxt < nchunk:
            read(nxt, slot)

    for i in range(max(0, nchunk - _NBUF), nchunk):
        wait_write(i, i % _NBUF)


def kernel(video_feats, video_masks, emb_table):
    B, L, D = video_feats.shape
    return pl.pallas_call(
        _body,
        in_specs=[
            pl.BlockSpec(memory_space=pl.ANY),
            pl.BlockSpec(memory_space=pl.ANY),
            pl.BlockSpec(memory_space=pl.ANY),
        ],
        out_specs=pl.BlockSpec(memory_space=pl.ANY),
        out_shape=jax.ShapeDtypeStruct((B, L, D), video_feats.dtype),
        scratch_shapes=[
            pltpu.VMEM((_NBUF, _C, L, D), jnp.float32),
            pltpu.VMEM((_NBUF, _C, L, D), jnp.float32),
            pltpu.VMEM((L, D), jnp.float32),
            pltpu.VMEM((B, L), jnp.float32),
            pltpu.SemaphoreType.DMA((_NBUF,)),
            pltpu.SemaphoreType.DMA((_NBUF,)),
            pltpu.SemaphoreType.DMA,
        ],
    )(video_feats, video_masks, emb_table)


# speedup vs baseline: 1.0172x; 1.0172x over previous
import jax
import jax.numpy as jnp
from jax.experimental import pallas as pl
from jax.experimental.pallas import tpu as pltpu

_CHUNKS = (8, 24, 32, 32, 32, 32, 32, 32, 24, 8)
_MAXC = max(_CHUNKS)
_NBUF = 3


def _body(f_hbm, m_hbm, e_hbm, o_hbm, in_buf, out_buf, pe_buf, mask_buf,
          in_sems, out_sems, aux_sem):
    B, L, D = f_hbm.shape
    offs = []
    o = 0
    for c in _CHUNKS:
        offs.append(o)
        o += c
    nchunk = len(_CHUNKS)

    emb_cp = pltpu.make_async_copy(e_hbm, pe_buf, aux_sem)
    emb_cp.start()
    mask_cp = pltpu.make_async_copy(m_hbm, mask_buf, aux_sem)
    mask_cp.start()
    emb_cp.wait()
    mask_cp.wait()
    pe = jnp.maximum(pe_buf[...], 0.0)

    def read(i, slot):
        pltpu.make_async_copy(
            f_hbm.at[pl.ds(offs[i], _CHUNKS[i])],
            in_buf.at[slot, pl.ds(0, _CHUNKS[i])], in_sems.at[slot]
        ).start()

    def wait_read(i, slot):
        pltpu.make_async_copy(
            f_hbm.at[pl.ds(offs[i], _CHUNKS[i])],
            in_buf.at[slot, pl.ds(0, _CHUNKS[i])], in_sems.at[slot]
        ).wait()

    def write(i, slot):
        pltpu.make_async_copy(
            out_buf.at[slot, pl.ds(0, _CHUNKS[i])],
            o_hbm.at[pl.ds(offs[i], _CHUNKS[i])], out_sems.at[slot]
        ).start()

    def wait_write(i, slot):
        pltpu.make_async_copy(
            out_buf.at[slot, pl.ds(0, _CHUNKS[i])],
            o_hbm.at[pl.ds(offs[i], _CHUNKS[i])], out_sems.at[slot]
        ).wait()

    for i in range(min(_NBUF, nchunk)):
        read(i, i % _NBUF)

    for i in range(nchunk):
        slot = i % _NBUF
        if i >= _NBUF:
            wait_write(i - _NBUF, slot)
        wait_read(i, slot)
        c = _CHUNKS[i]
        mk = mask_buf[pl.ds(offs[i], c), :]
        out_buf[slot, pl.ds(0, c)] = (
            in_buf[slot, pl.ds(0, c)] + pe[None, :, :] * mk[:, :, None])
        write(i, slot)
        nxt = i + _NBUF
        if nxt < nchunk:
            read(nxt, slot)

    for i in range(max(0, nchunk - _NBUF), nchunk):
        wait_write(i, i % _NBUF)


def kernel(video_feats, video_masks, emb_table):
    B, L, D = video_feats.shape
    return pl.pallas_call(
        _body,
        in_specs=[
            pl.BlockSpec(memory_space=pl.ANY),
            pl.BlockSpec(memory_space=pl.ANY),
            pl.BlockSpec(memory_space=pl.ANY),
        ],
        out_specs=pl.BlockSpec(memory_space=pl.ANY),
        out_shape=jax.ShapeDtypeStruct((B, L, D), video_feats.dtype),
        scratch_shapes=[
            pltpu.VMEM((_NBUF, _MAXC, L, D), jnp.float32),
            pltpu.VMEM((_NBUF, _MAXC, L, D), jnp.float32),
            pltpu.VMEM((L, D), jnp.float32),
            pltpu.VMEM((B, L), jnp.float32),
            pltpu.SemaphoreType.DMA((_NBUF,)),
            pltpu.SemaphoreType.DMA((_NBUF,)),
            pltpu.SemaphoreType.DMA,
        ],
    )(video_feats, video_masks, emb_table)
